# baseline (speedup 1.0000x reference)
import jax
import jax.numpy as jnp
from jax.experimental import pallas as pl
from jax.experimental.pallas import tpu as pltpu

N = 4096
BM = 256
BA = 512
_AR = 0.5
f32 = jnp.float32
bf16 = jnp.bfloat16
NB = N // BM


def _sig_t(z_blk, z_all):
    s = jax.lax.dot_general(z_blk * jnp.asarray(0.5, bf16), z_all,
                            (((1,), (1,)), ((), ())),
                            preferred_element_type=f32).astype(bf16)
    half = jnp.asarray(0.5, bf16)
    return half * jnp.tanh(s) + half


def _proj_body(x_ref, w_ref, o_ref):
    o_ref[...] = jnp.dot(x_ref[...], w_ref[...], preferred_element_type=f32)


def _proj(x, w):
    n, k = x.shape
    c = w.shape[1]
    return pl.pallas_call(
        _proj_body,
        grid=(n // BA,),
        in_specs=[pl.BlockSpec((BA, k), lambda i: (i, 0)),
                  pl.BlockSpec((k, c), lambda i: (0, 0))],
        out_specs=pl.BlockSpec((BA, c), lambda i: (i, 0)),
        out_shape=jax.ShapeDtypeStruct((n, c), f32),
    )(x, w)


def _adj_first_body(adj_ref, m_ref, o_ref, ab_ref):
    ab = adj_ref[...].astype(bf16)
    ab_ref[...] = ab
    acc = jnp.dot(ab, m_ref[...].astype(bf16), preferred_element_type=f32)
    o_ref[...] = jnp.maximum(acc, 0.0)


def _adj_first(adj, m):
    n = adj.shape[0]
    c = m.shape[1]
    return pl.pallas_call(
        _adj_first_body,
        grid=(n // BA,),
        in_specs=[pl.BlockSpec((BA, n), lambda i: (i, 0)),
                  pl.BlockSpec((n, c), lambda i: (0, 0))],
        out_specs=[pl.BlockSpec((BA, c), lambda i: (i, 0)),
                   pl.BlockSpec((BA, n), lambda i: (i, 0))],
        out_shape=[jax.ShapeDtypeStruct((n, c), f32),
                   jax.ShapeDtypeStruct((n, n), bf16)],
    )(adj, m)


def _mega_body(adj_ref, h1_ref, xw0_ref, h1r_ref, wm_ref,
               w1a_ref, w1b_ref, w2a_ref, w2b_ref, wh2_ref, wout_ref,
               ones_ref,
               out_ref, rec_ref,
               z_ref, rs_ref, m_ref, md_ref, vd_ref, u_ref, zf_ref, t3_ref):
    p = pl.program_id(0)
    i = pl.program_id(1)
    row = pl.ds(i * BM, BM)

    @pl.when(p == 0)
    def _():
        acc = jnp.dot(adj_ref[...], h1_ref[...].astype(bf16),
                      preferred_element_type=f32)
        z_ref[row, :] = jnp.dot(acc, wm_ref[...], preferred_element_type=f32)

    @pl.when(p == 1)
    def _():
        zi = z_ref[row, :]
        zb = z_ref[...].astype(bf16)
        sg = _sig_t(zi.astype(bf16), zb)
        rs_ref[row, :] = jnp.dot(sg, ones_ref[...],
                                 preferred_element_type=f32)[:, :1]
        m_ref[row, :] = jnp.concatenate(
            [jnp.dot(zi, w1a_ref[...], preferred_element_type=f32),
             xw0_ref[..., :32],
             jnp.dot(zi, w1b_ref[...], preferred_element_type=f32),
             xw0_ref[..., 32:]], axis=1)

    @pl.when(p == 2)
    def _():
        @pl.when(i == 0)
        def _():
            md_ref[...] = (m_ref[...] * jax.lax.rsqrt(rs_ref[...])).astype(bf16)

        sg = _sig_t(z_ref[row, :].astype(bf16), z_ref[...].astype(bf16))
        acc = jnp.dot(sg, md_ref[...], preferred_element_type=f32)
        di = jax.lax.rsqrt(rs_ref[row, :])
        sc = acc * di
        ua = jnp.maximum(sc[:, 0:32], 0.0) + jnp.maximum(sc[:, 32:64], 0.0)
        ub = jnp.maximum(sc[:, 64:96], 0.0) + jnp.maximum(sc[:, 96:128], 0.0)
        v = jnp.concatenate(
            [jnp.dot(ua, w2a_ref[...], preferred_element_type=f32),
             jnp.dot(ub, w2b_ref[...], preferred_element_type=f32)], axis=1)
        vd_ref[row, :] = (v * di).astype(bf16)

    @pl.when(p == 3)
    def _():
        zi = z_ref[row, :]
        sg = _sig_t(zi.astype(bf16), z_ref[...].astype(bf16))
        acc = jnp.dot(sg, vd_ref[...], preferred_element_type=f32)
        w = acc * jax.lax.rsqrt(rs_ref[row, :])
        upd = (1.0 - _AR) * jnp.concatenate([zi, zi], axis=1) + _AR * w
        u_ref[row, :] = upd[:, :16]
        zf_ref[row, :] = upd[:, 16:].astype(bf16)

    @pl.when(p == 4)
    def _():
        rec_ref[...] = jax.lax.dot_general(
            u_ref[row, :], u_ref[...], (((1,), (1,)), ((), ())),
            preferred_element_type=f32).reshape(BM * N)

    @pl.when(p == 5)
    def _():
        acc = jnp.dot(adj_ref[...], zf_ref[...], preferred_element_type=f32)
        acc = jnp.dot(acc, wh2_ref[...], preferred_element_type=f32)
        t3_ref[row, :] = (h1r_ref[...] + jnp.maximum(acc, 0.0)).astype(bf16)

    @pl.when(p == 6)
    def _():
        acc = jnp.dot(adj_ref[...], t3_ref[...], preferred_element_type=f32)
        out_ref[...] = jnp.dot(acc, wout_ref[...], preferred_element_type=f32)


def _adj_active(p):
    return (6 - p) // 6 + p // 5


def _mega(adj_b, hidden1, xw0, h1r, wm, w1a, w1b, w2a, w2b, wh2, wout):
    return pl.pallas_call(
        _mega_body,
        grid=(7, NB),
        in_specs=[
            pl.BlockSpec((BM, N), lambda p, i: (i * _adj_active(p), 0)),
            pl.BlockSpec((N, 32), lambda p, i: (0, 0)),
            pl.BlockSpec((BM, 64), lambda p, i: (i, 0)),
            pl.BlockSpec((BM, 32), lambda p, i: (i, 0)),
            pl.BlockSpec((32, 16), lambda p, i: (0, 0)),
            pl.BlockSpec((16, 32), lambda p, i: (0, 0)),
            pl.BlockSpec((16, 32), lambda p, i: (0, 0)),
            pl.BlockSpec((32, 16), lambda p, i: (0, 0)),
            pl.BlockSpec((32, 16), lambda p, i: (0, 0)),
            pl.BlockSpec((16, 32), lambda p, i: (0, 0)),
            pl.BlockSpec((32, 16), lambda p, i: (0, 0)),
            pl.BlockSpec((N, 8), lambda p, i: (0, 0)),
        ],
        out_specs=[
            pl.BlockSpec((BM, 16), lambda p, i: (i, 0)),
            pl.BlockSpec((BM * N,),
                         lambda p, i: (i * (p // 4 - p // 5)
                                       + (NB - 1) * (p // 5),)),
        ],
        out_shape=[jax.ShapeDtypeStruct((N, 16), f32),
                   jax.ShapeDtypeStruct((N * N,), f32)],
        scratch_shapes=[pltpu.VMEM((N, 16), f32),
                        pltpu.VMEM((N, 1), f32),
                        pltpu.VMEM((N, 128), f32),
                        pltpu.VMEM((N, 128), bf16),
                        pltpu.VMEM((N, 32), bf16),
                        pltpu.VMEM((N, 16), f32),
                        pltpu.VMEM((N, 16), bf16),
                        pltpu.VMEM((N, 32), bf16)],
    )(adj_b, hidden1, xw0, h1r, wm, w1a, w1b, w2a, w2b, wh2, wout,
      jnp.ones((N, 8), bf16))


def kernel(features, adj, W_e1, W_mean, W_std, Wl0_a, Wl1_a, Wl2_a,
           Wl0_b, Wl1_b, Wl2_b, W_h1, W_h2, W_out):
    wcat = jnp.concatenate([W_e1, W_h1, Wl0_a, Wl0_b], axis=1)
    p = _proj(features, wcat)

    t1, adj_b = _adj_first(adj, p[:, :64])
    hidden1, h1r = t1[:, :32], t1[:, 32:]

    outputs, reconstructions = _mega(
        adj_b, hidden1, p[:, 64:], h1r, W_mean,
        Wl1_a, Wl1_b, Wl2_a, Wl2_b, W_h2, W_out)
    return outputs, reconstructions


# speedup vs baseline: 1.1713x; 1.1713x over previous
import jax
import jax.numpy as jnp
from jax.experimental import pallas as pl
from jax.experimental.pallas import tpu as pltpu

N = 4096
BM = 256
_AR = 0.5
f32 = jnp.float32
bf16 = jnp.bfloat16
NB = N // BM


def _sig_t(z_blk, z_all):
    s = jax.lax.dot_general(z_blk * jnp.asarray(0.5, bf16), z_all,
                            (((1,), (1,)), ((), ())),
                            preferred_element_type=f32).astype(bf16)
    half = jnp.asarray(0.5, bf16)
    return half * jnp.tanh(s) + half


def _proj_body(x_ref, w_ref, o_ref):
    o_ref[...] = jnp.dot(x_ref[...], w_ref[...], preferred_element_type=f32)


def _proj(x, w):
    n, k = x.shape
    c = w.shape[1]
    return pl.pallas_call(
        _proj_body,
        grid=(n // 512,),
        in_specs=[pl.BlockSpec((512, k), lambda i: (i, 0)),
                  pl.BlockSpec((k, c), lambda i: (0, 0))],
        out_specs=pl.BlockSpec((512, c), lambda i: (i, 0)),
        out_shape=jax.ShapeDtypeStruct((n, c), f32),
    )(x, w)


def _mega_body(adj_ref, xp_ref, wm_ref,
               w1a_ref, w1b_ref, w2a_ref, w2b_ref, wh2_ref, wout_ref,
               ones_ref,
               out_ref, u_ref,
               adj_s, h1_ref, h1r_ref, z_ref, rs_ref, m_ref, md_ref,
               vd_ref, zf_ref, t3_ref):
    p = pl.program_id(0)
    i = pl.program_id(1)
    row = pl.ds(i * BM, BM)

    @pl.when(p == 0)
    def _():
        ab = adj_ref[...].astype(bf16)
        adj_s[row, :] = ab
        t1 = jnp.dot(ab, xp_ref[:, :64].astype(bf16),
                     preferred_element_type=f32)
        t1 = jnp.maximum(t1, 0.0)
        h1_ref[row, :] = t1[:, :32].astype(bf16)
        h1r_ref[row, :] = t1[:, 32:]

    @pl.when(p == 1)
    def _():
        acc = jnp.dot(adj_s[row, :], h1_ref[...], preferred_element_type=f32)
        z_ref[row, :] = jnp.dot(acc, wm_ref[...], preferred_element_type=f32)

    @pl.when(p == 2)
    def _():
        zi = z_ref[row, :]
        sg = _sig_t(zi.astype(bf16), z_ref[...].astype(bf16))
        rs_ref[row, :] = jnp.dot(sg, ones_ref[...],
                                 preferred_element_type=f32)[:, :1]
        m_ref[row, :] = jnp.concatenate(
            [jnp.dot(zi, w1a_ref[...], preferred_element_type=f32),
             xp_ref[row, 64:96],
             jnp.dot(zi, w1b_ref[...], preferred_element_type=f32),
             xp_ref[row, 96:128]], axis=1).astype(bf16)

    @pl.when(p == 3)
    def _():
        @pl.when(i == 0)
        def _():
            md_ref[...] = (m_ref[...].astype(f32)
                           * jax.lax.rsqrt(rs_ref[...])).astype(bf16)

        sg = _sig_t(z_ref[row, :].astype(bf16), z_ref[...].astype(bf16))
        acc = jnp.dot(sg, md_ref[...], preferred_element_type=f32)
        di = jax.lax.rsqrt(rs_ref[row, :])
        sc = acc * di
        ua = jnp.maximum(sc[:, 0:32], 0.0) + jnp.maximum(sc[:, 32:64], 0.0)
        ub = jnp.maximum(sc[:, 64:96], 0.0) + jnp.maximum(sc[:, 96:128], 0.0)
        v = jnp.concatenate(
            [jnp.dot(ua, w2a_ref[...], preferred_element_type=f32),
             jnp.dot(ub, w2b_ref[...], preferred_element_type=f32)], axis=1)
        vd_ref[row, :] = (v * di).astype(bf16)

    @pl.when(p == 4)
    def _():
        zi = z_ref[row, :]
        sg = _sig_t(zi.astype(bf16), z_ref[...].astype(bf16))
        acc = jnp.dot(sg, vd_ref[...], preferred_element_type=f32)
        w = acc * jax.lax.rsqrt(rs_ref[row, :])
        upd = (1.0 - _AR) * jnp.concatenate([zi, zi], axis=1) + _AR * w
        u_ref[...] = upd[:, :16]
        zf_ref[row, :] = upd[:, 16:].astype(bf16)

    @pl.when(p == 5)
    def _():
        acc = jnp.dot(adj_s[row, :], zf_ref[...], preferred_element_type=f32)
        acc = jnp.dot(acc, wh2_ref[...], preferred_element_type=f32)
        t3_ref[row, :] = (h1r_ref[row, :] + jnp.maximum(acc, 0.0)).astype(bf16)

    @pl.when(p == 6)
    def _():
        acc = jnp.dot(adj_s[row, :], t3_ref[...], preferred_element_type=f32)
        out_ref[...] = jnp.dot(acc, wout_ref[...], preferred_element_type=f32)


def _mega(adj, xp, wm, w1a, w1b, w2a, w2b, wh2, wout):
    return pl.pallas_call(
        _mega_body,
        grid=(7, NB),
        in_specs=[
            pl.BlockSpec((BM, N), lambda p, i: (i * ((6 - p) // 6), 0)),
            pl.BlockSpec((N, 128), lambda p, i: (0, 0)),
            pl.BlockSpec((32, 16), lambda p, i: (0, 0)),
            pl.BlockSpec((16, 32), lambda p, i: (0, 0)),
            pl.BlockSpec((16, 32), lambda p, i: (0, 0)),
            pl.BlockSpec((32, 16), lambda p, i: (0, 0)),
            pl.BlockSpec((32, 16), lambda p, i: (0, 0)),
            pl.BlockSpec((16, 32), lambda p, i: (0, 0)),
            pl.BlockSpec((32, 16), lambda p, i: (0, 0)),
            pl.BlockSpec((N, 8), lambda p, i: (0, 0)),
        ],
        out_specs=[
            pl.BlockSpec((BM, 16), lambda p, i: (i * (p // 6), 0)),
            pl.BlockSpec((BM, 16),
                         lambda p, i: (i * (p // 4 - p // 5)
                                       + (NB - 1) * (p // 5), 0)),
        ],
        out_shape=[jax.ShapeDtypeStruct((N, 16), f32),
                   jax.ShapeDtypeStruct((N, 16), f32)],
        scratch_shapes=[pltpu.VMEM((N, N), bf16),
                        pltpu.VMEM((N, 32), bf16),
                        pltpu.VMEM((N, 32), f32),
                        pltpu.VMEM((N, 16), f32),
                        pltpu.VMEM((N, 1), f32),
                        pltpu.VMEM((N, 128), bf16),
                        pltpu.VMEM((N, 128), bf16),
                        pltpu.VMEM((N, 32), bf16),
                        pltpu.VMEM((N, 16), bf16),
                        pltpu.VMEM((N, 32), bf16)],
        compiler_params=pltpu.CompilerParams(
            vmem_limit_bytes=100 * 1024 * 1024),
    )(adj, xp, wm, w1a, w1b, w2a, w2b, wh2, wout, jnp.ones((N, 8), bf16))


BR = 512


def _recon_body(u_blk, u_all, o_ref):
    o_ref[...] = jax.lax.dot_general(
        u_blk[...], u_all[...], (((1,), (1,)), ((), ())),
        preferred_element_type=f32).reshape(BR * N)


def _recon(u):
    return pl.pallas_call(
        _recon_body,
        grid=(N // BR,),
        in_specs=[pl.BlockSpec((BR, 16), lambda i: (i, 0)),
                  pl.BlockSpec((N, 16), lambda i: (0, 0))],
        out_specs=pl.BlockSpec((BR * N,), lambda i: (i,)),
        out_shape=jax.ShapeDtypeStruct((N * N,), f32),
    )(u, u)


def kernel(features, adj, W_e1, W_mean, W_std, Wl0_a, Wl1_a, Wl2_a,
           Wl0_b, Wl1_b, Wl2_b, W_h1, W_h2, W_out):
    wcat = jnp.concatenate([W_e1, W_h1, Wl0_a, Wl0_b], axis=1)
    xp = _proj(features, wcat)
    outputs, u_a = _mega(
        adj, xp, W_mean, Wl1_a, Wl1_b, Wl2_a, Wl2_b, W_h2, W_out)
    reconstructions = _recon(u_a)
    return outputs, reconstructions


# speedup vs baseline: 1.1843x; 1.0111x over previous
import jax
import jax.numpy as jnp
from jax.experimental import pallas as pl
from jax.experimental.pallas import tpu as pltpu

N = 4096
BM = 256
_AR = 0.5
f32 = jnp.float32
bf16 = jnp.bfloat16
NB = N // BM


def _sig_t(z_blk, z_all):
    s = jax.lax.dot_general(z_blk * jnp.asarray(0.5, bf16), z_all,
                            (((1,), (1,)), ((), ())),
                            preferred_element_type=f32).astype(bf16)
    half = jnp.asarray(0.5, bf16)
    return half * jnp.tanh(s) + half


def _proj_body(x_ref, w_ref, o_ref):
    o_ref[...] = jnp.dot(x_ref[...], w_ref[...], preferred_element_type=f32)


def _proj(x, w):
    n, k = x.shape
    c = w.shape[1]
    return pl.pallas_call(
        _proj_body,
        grid=(n // 512,),
        in_specs=[pl.BlockSpec((512, k), lambda i: (i, 0)),
                  pl.BlockSpec((k, c), lambda i: (0, 0))],
        out_specs=pl.BlockSpec((512, c), lambda i: (i, 0)),
        out_shape=jax.ShapeDtypeStruct((n, c), f32),
    )(x, w)


def _mega_body(adj_ref, xp_ref, wm_ref,
               w1a_ref, w1b_ref, w2a_ref, w2b_ref, wh2_ref, wout_ref,
               ones_ref,
               out_ref, u_ref,
               adj_s, h1_ref, h1r_ref, z_ref, rs_ref, m_ref, md_ref,
               vd_ref, zf_ref, t3_ref, hw_ref, tw_ref):
    p = pl.program_id(0)
    i = pl.program_id(1)
    row = pl.ds(i * BM, BM)

    @pl.when(p == 0)
    def _():
        ab = adj_ref[...].astype(bf16)
        adj_s[row, :] = ab
        t1 = jnp.dot(ab, xp_ref[:, :64].astype(bf16),
                     preferred_element_type=f32)
        t1 = jnp.maximum(t1, 0.0)
        h1_ref[row, :] = t1[:, :32].astype(bf16)
        h1r_ref[row, :] = t1[:, 32:]

    @pl.when(p == 1)
    def _():
        @pl.when(i == 0)
        def _():
            hw_ref[...] = jnp.dot(h1_ref[...], wm_ref[...].astype(bf16),
                                  preferred_element_type=f32).astype(bf16)

        z_ref[row, :] = jnp.dot(adj_s[row, :], hw_ref[...],
                                preferred_element_type=f32)

    @pl.when(p == 2)
    def _():
        zi = z_ref[row, :]
        sg = _sig_t(zi.astype(bf16), z_ref[...].astype(bf16))
        rs_ref[row, :] = jnp.dot(sg, ones_ref[...],
                                 preferred_element_type=f32)[:, :1]
        m_ref[row, :] = jnp.concatenate(
            [jnp.dot(zi, w1a_ref[...], preferred_element_type=f32),
             xp_ref[row, 64:96],
             jnp.dot(zi, w1b_ref[...], preferred_element_type=f32),
             xp_ref[row, 96:128]], axis=1).astype(bf16)

    @pl.when(p == 3)
    def _():
        @pl.when(i == 0)
        def _():
            md_ref[...] = (m_ref[...].astype(f32)
                           * jax.lax.rsqrt(rs_ref[...])).astype(bf16)

        sg = _sig_t(z_ref[row, :].astype(bf16), z_ref[...].astype(bf16))
        acc = jnp.dot(sg, md_ref[...], preferred_element_type=f32)
        di = jax.lax.rsqrt(rs_ref[row, :])
        sc = acc * di
        ua = jnp.maximum(sc[:, 0:32], 0.0) + jnp.maximum(sc[:, 32:64], 0.0)
        ub = jnp.maximum(sc[:, 64:96], 0.0) + jnp.maximum(sc[:, 96:128], 0.0)
        v = jnp.concatenate(
            [jnp.dot(ua, w2a_ref[...], preferred_element_type=f32),
             jnp.dot(ub, w2b_ref[...], preferred_element_type=f32)], axis=1)
        vd_ref[row, :] = (v * di).astype(bf16)

    @pl.when(p == 4)
    def _():
        zi = z_ref[row, :]
        sg = _sig_t(zi.astype(bf16), z_ref[...].astype(bf16))
        acc = jnp.dot(sg, vd_ref[...], preferred_element_type=f32)
        w = acc * jax.lax.rsqrt(rs_ref[row, :])
        upd = (1.0 - _AR) * jnp.concatenate([zi, zi], axis=1) + _AR * w
        u_ref[...] = upd[:, :16]
        zf_ref[row, :] = upd[:, 16:].astype(bf16)

    @pl.when(p == 5)
    def _():
        acc = jnp.dot(adj_s[row, :], zf_ref[...], preferred_element_type=f32)
        acc = jnp.dot(acc, wh2_ref[...], preferred_element_type=f32)
        t3_ref[row, :] = (h1r_ref[row, :] + jnp.maximum(acc, 0.0)).astype(bf16)

    @pl.when(p == 6)
    def _():
        @pl.when(i == 0)
        def _():
            tw_ref[...] = jnp.dot(t3_ref[...], wout_ref[...].astype(bf16),
                                  preferred_element_type=f32).astype(bf16)

        out_ref[...] = jnp.dot(adj_s[row, :], tw_ref[...],
                               preferred_element_type=f32)


def _mega(adj, xp, wm, w1a, w1b, w2a, w2b, wh2, wout):
    return pl.pallas_call(
        _mega_body,
        grid=(7, NB),
        in_specs=[
            pl.BlockSpec((BM, N), lambda p, i: (i * ((6 - p) // 6), 0)),
            pl.BlockSpec((N, 128), lambda p, i: (0, 0)),
            pl.BlockSpec((32, 16), lambda p, i: (0, 0)),
            pl.BlockSpec((16, 32), lambda p, i: (0, 0)),
            pl.BlockSpec((16, 32), lambda p, i: (0, 0)),
            pl.BlockSpec((32, 16), lambda p, i: (0, 0)),
            pl.BlockSpec((32, 16), lambda p, i: (0, 0)),
            pl.BlockSpec((16, 32), lambda p, i: (0, 0)),
            pl.BlockSpec((32, 16), lambda p, i: (0, 0)),
            pl.BlockSpec((N, 8), lambda p, i: (0, 0)),
        ],
        out_specs=[
            pl.BlockSpec((BM, 16), lambda p, i: (i * (p // 6), 0)),
            pl.BlockSpec((BM, 16),
                         lambda p, i: (i * (p // 4 - p // 5)
                                       + (NB - 1) * (p // 5), 0)),
        ],
        out_shape=[jax.ShapeDtypeStruct((N, 16), f32),
                   jax.ShapeDtypeStruct((N, 16), f32)],
        scratch_shapes=[pltpu.VMEM((N, N), bf16),
                        pltpu.VMEM((N, 32), bf16),
                        pltpu.VMEM((N, 32), f32),
                        pltpu.VMEM((N, 16), f32),
                        pltpu.VMEM((N, 1), f32),
                        pltpu.VMEM((N, 128), bf16),
                        pltpu.VMEM((N, 128), bf16),
                        pltpu.VMEM((N, 32), bf16),
                        pltpu.VMEM((N, 16), bf16),
                        pltpu.VMEM((N, 32), bf16),
                        pltpu.VMEM((N, 16), bf16),
                        pltpu.VMEM((N, 16), bf16)],
        compiler_params=pltpu.CompilerParams(
            vmem_limit_bytes=100 * 1024 * 1024),
    )(adj, xp, wm, w1a, w1b, w2a, w2b, wh2, wout, jnp.ones((N, 8), bf16))


BR = 512


def _recon_body(u_blk, u_all, o_ref):
    o_ref[...] = jax.lax.dot_general(
        u_blk[...], u_all[...], (((1,), (1,)), ((), ())),
        preferred_element_type=f32).reshape(BR * N)


def _recon(u):
    return pl.pallas_call(
        _recon_body,
        grid=(N // BR,),
        in_specs=[pl.BlockSpec((BR, 16), lambda i: (i, 0)),
                  pl.BlockSpec((N, 16), lambda i: (0, 0))],
        out_specs=pl.BlockSpec((BR * N,), lambda i: (i,)),
        out_shape=jax.ShapeDtypeStruct((N * N,), f32),
    )(u, u)


def kernel(features, adj, W_e1, W_mean, W_std, Wl0_a, Wl1_a, Wl2_a,
           Wl0_b, Wl1_b, Wl2_b, W_h1, W_h2, W_out):
    wcat = jnp.concatenate([W_e1, W_h1, Wl0_a, Wl0_b], axis=1)
    xp = _proj(features, wcat)
    outputs, u_a = _mega(
        adj, xp, W_mean, Wl1_a, Wl1_b, Wl2_a, Wl2_b, W_h2, W_out)
    reconstructions = _recon(u_a)
    return outputs, reconstructions
